# baseline (speedup 1.0000x reference)
import functools
import jax
import jax.numpy as jnp
import numpy as np
from jax import lax
from jax.experimental import pallas as pl
from jax.experimental.pallas import tpu as pltpu
from jax.experimental.pallas import tpu_sc as plsc

B, N, NQ, C, Ns, H, W = 2, 6, 2500, 256, 8, 64, 64
BN = B * N
V = BN * H * W
G = N * Ns * 4
QB = 256
NQPAD = 2560
BTOT = B * NQPAD
NWORK = 32
QPW = BTOT // NWORK
QCHUNK = 16


def _fuse_body(w_aug_ref, op_t_ref, vp_aug_ref, t1_ref, g_ref):
    t1 = jnp.dot(w_aug_ref[...], op_t_ref[...], preferred_element_type=jnp.float32)
    t1_ref[...] = t1
    g_ref[...] = jnp.dot(vp_aug_ref[...], t1[0:C, :], preferred_element_type=jnp.float32)


def _fuse_weights(w_aug, op_t, vp_aug):
    return pl.pallas_call(
        _fuse_body,
        out_shape=(
            jax.ShapeDtypeStruct((264, C), jnp.float32),
            jax.ShapeDtypeStruct((264, C), jnp.float32),
        ),
    )(w_aug, op_t, vp_aug)


def _table_body(x_ref, ga_ref, o_ref):
    o_ref[...] = (
        jnp.dot(x_ref[...], ga_ref[0:C, :], preferred_element_type=jnp.float32)
        + ga_ref[C:C + 1, :]
    ).astype(jnp.bfloat16)


def _make_table(x, g_aug):
    blk = 1024
    return pl.pallas_call(
        _table_body,
        grid=(V // blk,),
        in_specs=[
            pl.BlockSpec((blk, C), lambda i: (i, 0)),
            pl.BlockSpec((264, C), lambda i: (0, 0)),
        ],
        out_specs=pl.BlockSpec((blk, C), lambda i: (i, 0)),
        out_shape=jax.ShapeDtypeStruct((V, C), jnp.bfloat16),
    )(x, g_aug)


def _prep_body(q_ref, refx_ref, refy_ref, mask_ref, dpx_w_ref, dpy_w_ref,
               a_w_ref, dpb_ref, ab_ref, idx_ref, wgt_ref, r_ref):
    b = pl.program_id(0)
    qb = q_ref[0]
    dpx = jnp.dot(dpx_w_ref[...], qb, preferred_element_type=jnp.float32)
    dpx = dpx + dpb_ref[0:Ns, 0:1]
    dpy = jnp.dot(dpy_w_ref[...], qb, preferred_element_type=jnp.float32)
    dpy = dpy + dpb_ref[Ns:2 * Ns, 0:1]
    logits = jnp.dot(a_w_ref[...], qb, preferred_element_type=jnp.float32)
    logits = logits + ab_ref[:, 0:1]
    mx = jnp.max(logits, axis=0, keepdims=True)
    ex = jnp.exp(logits - mx)
    attn = ex / jnp.sum(ex, axis=0, keepdims=True)

    msum = jnp.sum(mask_ref[0], axis=0, keepdims=True)
    r = msum / (msum + 1e-6)
    r_ref[0] = jnp.broadcast_to(r, (16, r.shape[1]))
    inv_m = 1.0 / (msum + 1e-6)

    for n in range(N):
        mrow = mask_ref[0, n:n + 1, :]
        px = (refx_ref[0, n:n + 1, :] + dpx) * (W - 1.0)
        py = (refy_ref[0, n:n + 1, :] + dpy) * (H - 1.0)
        x0 = jnp.floor(px)
        y0 = jnp.floor(py)
        fx = px - x0
        fy = py - y0
        wq = mrow * attn * inv_m
        base = (b * N + n) * (H * W)
        ci = 0
        for dy, wyf in ((0, 1.0 - fy), (1, fy)):
            for dx, wxf in ((0, 1.0 - fx), (1, fx)):
                xi = x0 + dx
                yi = y0 + dy
                valid = ((xi >= 0.0) & (xi <= W - 1.0)
                         & (yi >= 0.0) & (yi <= H - 1.0))
                xc = jnp.clip(xi, 0.0, W - 1.0).astype(jnp.int32)
                yc = jnp.clip(yi, 0.0, H - 1.0).astype(jnp.int32)
                sub = n * (4 * Ns) + ci * Ns
                idx_ref[0, sub:sub + Ns, :] = base + yc * W + xc
                wgt_ref[0, sub:sub + Ns, :] = wq * wxf * wyf * valid.astype(jnp.float32)
                ci += 1


def _prep(q_t, refx, refy, mask_t, dpx_w, dpy_w, a_w, dpb, ab):
    nb = NQPAD // QB
    return pl.pallas_call(
        _prep_body,
        grid=(B, nb),
        in_specs=[
            pl.BlockSpec((1, C, QB), lambda b, j: (b, 0, j)),
            pl.BlockSpec((1, 8, QB), lambda b, j: (b, 0, j)),
            pl.BlockSpec((1, 8, QB), lambda b, j: (b, 0, j)),
            pl.BlockSpec((1, 8, QB), lambda b, j: (b, 0, j)),
            pl.BlockSpec((Ns, C), lambda b, j: (0, 0)),
            pl.BlockSpec((Ns, C), lambda b, j: (0, 0)),
            pl.BlockSpec((Ns, C), lambda b, j: (0, 0)),
            pl.BlockSpec((2 * Ns, 128), lambda b, j: (0, 0)),
            pl.BlockSpec((Ns, 128), lambda b, j: (0, 0)),
        ],
        out_specs=[
            pl.BlockSpec((1, G, QB), lambda b, j: (b, 0, j)),
            pl.BlockSpec((1, G, QB), lambda b, j: (b, 0, j)),
            pl.BlockSpec((1, 16, QB), lambda b, j: (b, 0, j)),
        ],
        out_shape=(
            jax.ShapeDtypeStruct((B, G, NQPAD), jnp.int32),
            jax.ShapeDtypeStruct((B, G, NQPAD), jnp.float32),
            jax.ShapeDtypeStruct((B, 16, NQPAD), jnp.float32),
        ),
    )(q_t, refx, refy, mask_t, dpx_w, dpy_w, a_w, dpb, ab)


NCHUNK = 16


def _sc_body(table_hbm, idx_hbm, wgt_hbm, r_hbm, opb_hbm, gv_hbm, out_hbm,
             meta_i, meta_w, meta_r, rows0, rows1, acc0, acc1, opb_v, gv_v,
             sem_g0, sem_g1, sem_s0, sem_s1):
    wid = lax.axis_index("s") * 2 + lax.axis_index("c")
    qbase = wid * QPW
    pltpu.sync_copy(opb_hbm, opb_v)
    pltpu.sync_copy(gv_hbm, gv_v)

    def copy_meta(i):
        blk = i >> 4
        slot = blk & 1
        q0 = qbase + blk * QCHUNK
        pltpu.sync_copy(idx_hbm.at[pl.ds(q0, QCHUNK)], meta_i.at[slot])
        pltpu.sync_copy(wgt_hbm.at[pl.ds(q0, QCHUNK)], meta_w.at[slot])
        pltpu.sync_copy(r_hbm.at[pl.ds(q0, QCHUNK)], meta_r.at[slot])

    def gather_cps(i, rows, sem):
        slot = (i >> 4) & 1
        mi = i & 15
        cpa = pltpu.make_async_copy(table_hbm.at[meta_i.at[slot, mi, 0]],
                                    rows.at[pl.ds(0, G // 2)], sem)
        cpb = pltpu.make_async_copy(table_hbm.at[meta_i.at[slot, mi, 1]],
                                    rows.at[pl.ds(G // 2, G // 2)], sem)
        return cpa, cpb

    def reduce_to(i, rows, acc):
        slot = (i >> 4) & 1
        mi = i & 15

        def red(j, accs):
            wvec = meta_w[slot, mi, pl.ds(j * 16, 16)]
            rbase = j * 16
            for e in range(16):
                we = wvec[e]
                new = list(accs)
                for grp in range(NCHUNK // 2):
                    v = rows[rbase + e, pl.ds(grp * 16, 16)]
                    lo = lax.bitcast_convert_type(v << 16, jnp.float32)
                    hi = lax.bitcast_convert_type(v & jnp.int32(-65536),
                                                  jnp.float32)
                    new[2 * grp] = new[2 * grp] + lo * we
                    new[2 * grp + 1] = new[2 * grp + 1] + hi * we
                accs = tuple(new)
            return accs

        accs = lax.fori_loop(
            0, G // 16, red,
            tuple(jnp.zeros((16,), jnp.float32) for _ in range(NCHUNK)))
        rv = meta_r[slot, mi, pl.ds(0, 16)][0]
        for c in range(NCHUNK):
            sl = pl.ds(c * 16, 16)
            acc[sl] = accs[c] + opb_v[sl] + gv_v[sl] * rv

    copy_meta(0)
    pa, pb = gather_cps(0, rows0, sem_g0)
    pa.start()
    pb.start()

    nk = QPW // 2

    def k_body(k, _):
        i0 = 2 * k
        i1 = 2 * k + 1
        wa, wb = gather_cps(i0, rows0, sem_g0)
        wa.wait()
        wb.wait()
        c1a, c1b = gather_cps(i1, rows1, sem_g1)
        c1a.start()
        c1b.start()
        reduce_to(i0, rows0, acc0)

        @pl.when(k > 0)
        def _w0():
            pltpu.make_async_copy(acc0, out_hbm.at[qbase + i0 - 2],
                                  sem_s0).wait()

        pltpu.make_async_copy(acc0, out_hbm.at[qbase + i0], sem_s0).start()

        c1a.wait()
        c1b.wait()

        @pl.when(k < nk - 1)
        def _nx():
            inext = i0 + 2

            @pl.when((inext & 15) == 0)
            def _cm():
                copy_meta(inext)

            na, nb = gather_cps(inext, rows0, sem_g0)
            na.start()
            nb.start()

        reduce_to(i1, rows1, acc1)

        @pl.when(k > 0)
        def _w1():
            pltpu.make_async_copy(acc1, out_hbm.at[qbase + i1 - 2],
                                  sem_s1).wait()

        pltpu.make_async_copy(acc1, out_hbm.at[qbase + i1], sem_s1).start()
        return _

    lax.fori_loop(0, nk, k_body, 0)
    pltpu.make_async_copy(acc0, out_hbm.at[qbase + QPW - 2], sem_s0).wait()
    pltpu.make_async_copy(acc1, out_hbm.at[qbase + QPW - 1], sem_s1).wait()


def _sc_gather_reduce(table, idx_q, wgt_q, r_q, op_b, gvec):
    kern = pl.kernel(
        _sc_body,
        out_type=jax.ShapeDtypeStruct((BTOT, C), jnp.float32),
        mesh=plsc.VectorSubcoreMesh(core_axis_name="c", subcore_axis_name="s"),
        scratch_types=[
            pltpu.VMEM((2, QCHUNK, 2, G // 2), jnp.int32),
            pltpu.VMEM((2, QCHUNK, G), jnp.float32),
            pltpu.VMEM((2, QCHUNK, 16), jnp.float32),
            pltpu.VMEM((G, C // 2), jnp.int32),
            pltpu.VMEM((G, C // 2), jnp.int32),
            pltpu.VMEM((C,), jnp.float32),
            pltpu.VMEM((C,), jnp.float32),
            pltpu.VMEM((C,), jnp.float32),
            pltpu.VMEM((C,), jnp.float32),
            pltpu.SemaphoreType.DMA,
            pltpu.SemaphoreType.DMA,
            pltpu.SemaphoreType.DMA,
            pltpu.SemaphoreType.DMA,
        ],
    )
    return kern(table, idx_q, wgt_q, r_q, op_b, gvec)


@jax.jit
def kernel(query, ref_points, image_features, mask, dp_w, dp_b, A_w, A_b,
           W_w, W_b, vp_w, vp_b, op_w, op_b):
    f32 = jnp.float32

    w_aug = jnp.pad(jnp.concatenate([W_w.T, W_b[None, :]], axis=0),
                    ((0, 7), (0, 0)))
    vp_aug = jnp.pad(jnp.concatenate([vp_w.T, vp_b[None, :]], axis=0),
                     ((0, 7), (0, 0)))
    t1_aug, g_aug = _fuse_weights(w_aug, op_w.T, vp_aug)
    gvec = t1_aug[C]

    perm = np.arange(C).reshape(C // 32, 2, 16)
    perm = np.stack([perm[:, 0], perm[:, 1]], axis=-1).reshape(C)
    x = image_features.transpose(0, 2, 3, 1).reshape(V, C)
    table = _make_table(x, g_aug[:, perm])
    table = lax.bitcast_convert_type(table.reshape(V, C // 2, 2), jnp.int32)

    q_t = jnp.pad(query.transpose(0, 2, 1), ((0, 0), (0, 0), (0, NQPAD - NQ)))
    refx = jnp.pad(ref_points[..., 0], ((0, 0), (0, 2), (0, NQPAD - NQ)))
    refy = jnp.pad(ref_points[..., 1], ((0, 0), (0, 2), (0, NQPAD - NQ)))
    mask_t = jnp.pad(mask, ((0, 0), (0, 2), (0, NQPAD - NQ)))
    dpx_w = dp_w[0::2]
    dpy_w = dp_w[1::2]
    dpb = jnp.broadcast_to(
        jnp.concatenate([dp_b[0::2], dp_b[1::2]])[:, None], (2 * Ns, 128))
    ab = jnp.broadcast_to(A_b[:, None], (Ns, 128))

    idx_t, wgt_t, r_t = _prep(q_t, refx, refy, mask_t, dpx_w, dpy_w, A_w,
                              dpb, ab)

    idx_q = idx_t.transpose(0, 2, 1).reshape(BTOT, 2, G // 2)
    wgt_q = wgt_t.transpose(0, 2, 1).reshape(BTOT, G)
    r_q = r_t.transpose(0, 2, 1).reshape(BTOT, 16)

    z = _sc_gather_reduce(table, idx_q, wgt_q, r_q,
                          op_b.astype(f32), gvec)
    return z.reshape(B, NQPAD, C)[:, :NQ]


# speedup vs baseline: 1.4887x; 1.4887x over previous
import functools
import jax
import jax.numpy as jnp
import numpy as np
from jax import lax
from jax.experimental import pallas as pl
from jax.experimental.pallas import tpu as pltpu
from jax.experimental.pallas import tpu_sc as plsc

B, N, NQ, C, Ns, H, W = 2, 6, 2500, 256, 8, 64, 64
BN = B * N
V = BN * H * W
G = N * Ns * 4
QB = 256
NQPAD = 2560
BTOT = B * NQPAD
NWORK = 32
QPW = BTOT // NWORK
QCHUNK = 16


def _fuse_body(w_aug_ref, op_t_ref, vp_aug_ref, t1_ref, g_ref):
    t1 = jnp.dot(w_aug_ref[...], op_t_ref[...], preferred_element_type=jnp.float32)
    t1_ref[...] = t1
    g_ref[...] = jnp.dot(vp_aug_ref[...], t1[0:C, :], preferred_element_type=jnp.float32)


def _fuse_weights(w_aug, op_t, vp_aug):
    return pl.pallas_call(
        _fuse_body,
        out_shape=(
            jax.ShapeDtypeStruct((264, C), jnp.float32),
            jax.ShapeDtypeStruct((264, C), jnp.float32),
        ),
    )(w_aug, op_t, vp_aug)


def _table_body(x_ref, ga_ref, o_ref):
    o_ref[...] = (
        jnp.dot(x_ref[...], ga_ref[0:C, :], preferred_element_type=jnp.float32)
        + ga_ref[C:C + 1, :]
    ).astype(jnp.bfloat16)


def _make_table(x, g_aug):
    blk = 1024
    return pl.pallas_call(
        _table_body,
        grid=(V // blk,),
        in_specs=[
            pl.BlockSpec((blk, C), lambda i: (i, 0)),
            pl.BlockSpec((264, C), lambda i: (0, 0)),
        ],
        out_specs=pl.BlockSpec((blk, C), lambda i: (i, 0)),
        out_shape=jax.ShapeDtypeStruct((V, C), jnp.bfloat16),
    )(x, g_aug)


def _prep_body(q_ref, refx_ref, refy_ref, mask_ref, dpx_w_ref, dpy_w_ref,
               a_w_ref, dpb_ref, ab_ref, idx_ref, wgt_ref, r_ref):
    b = pl.program_id(0)
    qb = q_ref[0]
    dpx = jnp.dot(dpx_w_ref[...], qb, preferred_element_type=jnp.float32)
    dpx = dpx + dpb_ref[0:Ns, 0:1]
    dpy = jnp.dot(dpy_w_ref[...], qb, preferred_element_type=jnp.float32)
    dpy = dpy + dpb_ref[Ns:2 * Ns, 0:1]
    logits = jnp.dot(a_w_ref[...], qb, preferred_element_type=jnp.float32)
    logits = logits + ab_ref[:, 0:1]
    mx = jnp.max(logits, axis=0, keepdims=True)
    ex = jnp.exp(logits - mx)
    attn = ex / jnp.sum(ex, axis=0, keepdims=True)

    msum = jnp.sum(mask_ref[0], axis=0, keepdims=True)
    r = msum / (msum + 1e-6)
    r_ref[0] = jnp.broadcast_to(r, (16, r.shape[1]))
    inv_m = 1.0 / (msum + 1e-6)

    for n in range(N):
        mrow = mask_ref[0, n:n + 1, :]
        px = (refx_ref[0, n:n + 1, :] + dpx) * (W - 1.0)
        py = (refy_ref[0, n:n + 1, :] + dpy) * (H - 1.0)
        x0 = jnp.floor(px)
        y0 = jnp.floor(py)
        fx = px - x0
        fy = py - y0
        wq = mrow * attn * inv_m
        base = (b * N + n) * (H * W)
        ci = 0
        for dy, wyf in ((0, 1.0 - fy), (1, fy)):
            for dx, wxf in ((0, 1.0 - fx), (1, fx)):
                xi = x0 + dx
                yi = y0 + dy
                valid = ((xi >= 0.0) & (xi <= W - 1.0)
                         & (yi >= 0.0) & (yi <= H - 1.0))
                xc = jnp.clip(xi, 0.0, W - 1.0).astype(jnp.int32)
                yc = jnp.clip(yi, 0.0, H - 1.0).astype(jnp.int32)
                sub = n * (4 * Ns) + ci * Ns
                idx_ref[0, sub:sub + Ns, :] = base + yc * W + xc
                wgt_ref[0, sub:sub + Ns, :] = wq * wxf * wyf * valid.astype(jnp.float32)
                ci += 1


def _prep(q_t, refx, refy, mask_t, dpx_w, dpy_w, a_w, dpb, ab):
    nb = NQPAD // QB
    return pl.pallas_call(
        _prep_body,
        grid=(B, nb),
        in_specs=[
            pl.BlockSpec((1, C, QB), lambda b, j: (b, 0, j)),
            pl.BlockSpec((1, 8, QB), lambda b, j: (b, 0, j)),
            pl.BlockSpec((1, 8, QB), lambda b, j: (b, 0, j)),
            pl.BlockSpec((1, 8, QB), lambda b, j: (b, 0, j)),
            pl.BlockSpec((Ns, C), lambda b, j: (0, 0)),
            pl.BlockSpec((Ns, C), lambda b, j: (0, 0)),
            pl.BlockSpec((Ns, C), lambda b, j: (0, 0)),
            pl.BlockSpec((2 * Ns, 128), lambda b, j: (0, 0)),
            pl.BlockSpec((Ns, 128), lambda b, j: (0, 0)),
        ],
        out_specs=[
            pl.BlockSpec((1, G, QB), lambda b, j: (b, 0, j)),
            pl.BlockSpec((1, G, QB), lambda b, j: (b, 0, j)),
            pl.BlockSpec((1, 16, QB), lambda b, j: (b, 0, j)),
        ],
        out_shape=(
            jax.ShapeDtypeStruct((B, G, NQPAD), jnp.int32),
            jax.ShapeDtypeStruct((B, G, NQPAD), jnp.float32),
            jax.ShapeDtypeStruct((B, 16, NQPAD), jnp.float32),
        ),
    )(q_t, refx, refy, mask_t, dpx_w, dpy_w, a_w, dpb, ab)


NCHUNK = 16


def _sc_body(table_hbm, idx_hbm, wgt_hbm, r_hbm, opb_hbm, gv_hbm, out_hbm,
             meta_i, meta_w, meta_r, rows0, rows1, acc0, acc1, opb_v, gv_v,
             sem_g0, sem_g1, sem_s0, sem_s1):
    wid = lax.axis_index("s") * 2 + lax.axis_index("c")
    qbase = wid * QPW
    pltpu.sync_copy(opb_hbm, opb_v)
    pltpu.sync_copy(gv_hbm, gv_v)

    def copy_meta(i):
        blk = i >> 4
        slot = blk & 1
        q0 = qbase + blk * QCHUNK
        pltpu.sync_copy(idx_hbm.at[pl.ds(q0, QCHUNK)], meta_i.at[slot])
        pltpu.sync_copy(wgt_hbm.at[pl.ds(q0, QCHUNK)], meta_w.at[slot])
        pltpu.sync_copy(r_hbm.at[pl.ds(q0, QCHUNK)], meta_r.at[slot])

    def gather_cps(i, rows, sem):
        slot = (i >> 4) & 1
        mi = i & 15
        cpa = pltpu.make_async_copy(table_hbm.at[meta_i.at[slot, mi, 0]],
                                    rows.at[pl.ds(0, G // 2)], sem)
        cpb = pltpu.make_async_copy(table_hbm.at[meta_i.at[slot, mi, 1]],
                                    rows.at[pl.ds(G // 2, G // 2)], sem)
        return cpa, cpb

    def reduce_to(i, rows, acc):
        slot = (i >> 4) & 1
        mi = i & 15
        rv = meta_r[slot, mi, pl.ds(0, 16)][0]
        for h in range(2):

            def red(j, accs):
                wvec = meta_w[slot, mi, pl.ds(j * 16, 16)]
                rbase = j * 16
                for e in range(16):
                    we = wvec[e]
                    new = list(accs)
                    for g4 in range(4):
                        grp = h * 4 + g4
                        v = rows[rbase + e, pl.ds(grp * 16, 16)]
                        lo = lax.bitcast_convert_type(v << 16, jnp.float32)
                        hi = lax.bitcast_convert_type(
                            v & jnp.int32(-65536), jnp.float32)
                        new[2 * g4] = new[2 * g4] + lo * we
                        new[2 * g4 + 1] = new[2 * g4 + 1] + hi * we
                    accs = tuple(new)
                return accs

            accs = lax.fori_loop(
                0, G // 16, red,
                tuple(jnp.zeros((16,), jnp.float32) for _ in range(8)))
            for g4 in range(4):
                grp = h * 4 + g4
                for half in range(2):
                    sl = pl.ds(grp * 32 + half * 16, 16)
                    acc[sl] = (accs[2 * g4 + half] + opb_v[sl]
                               + gv_v[sl] * rv)

    copy_meta(0)
    pa, pb = gather_cps(0, rows0, sem_g0)
    pa.start()
    pb.start()

    nk = QPW // 2

    def k_body(k, _):
        i0 = 2 * k
        i1 = 2 * k + 1
        wa, wb = gather_cps(i0, rows0, sem_g0)
        wa.wait()
        wb.wait()
        c1a, c1b = gather_cps(i1, rows1, sem_g1)
        c1a.start()
        c1b.start()
        reduce_to(i0, rows0, acc0)

        @pl.when(k > 0)
        def _w0():
            pltpu.make_async_copy(acc0, out_hbm.at[qbase + i0 - 2],
                                  sem_s0).wait()

        pltpu.make_async_copy(acc0, out_hbm.at[qbase + i0], sem_s0).start()

        c1a.wait()
        c1b.wait()

        @pl.when(k < nk - 1)
        def _nx():
            inext = i0 + 2

            @pl.when((inext & 15) == 0)
            def _cm():
                copy_meta(inext)

            na, nb = gather_cps(inext, rows0, sem_g0)
            na.start()
            nb.start()

        reduce_to(i1, rows1, acc1)

        @pl.when(k > 0)
        def _w1():
            pltpu.make_async_copy(acc1, out_hbm.at[qbase + i1 - 2],
                                  sem_s1).wait()

        pltpu.make_async_copy(acc1, out_hbm.at[qbase + i1], sem_s1).start()
        return _

    lax.fori_loop(0, nk, k_body, 0)
    pltpu.make_async_copy(acc0, out_hbm.at[qbase + QPW - 2], sem_s0).wait()
    pltpu.make_async_copy(acc1, out_hbm.at[qbase + QPW - 1], sem_s1).wait()


def _sc_gather_reduce(table, idx_q, wgt_q, r_q, op_b, gvec):
    kern = pl.kernel(
        _sc_body,
        out_type=jax.ShapeDtypeStruct((BTOT, C), jnp.float32),
        mesh=plsc.VectorSubcoreMesh(core_axis_name="c", subcore_axis_name="s"),
        scratch_types=[
            pltpu.VMEM((2, QCHUNK, 2, G // 2), jnp.int32),
            pltpu.VMEM((2, QCHUNK, G), jnp.float32),
            pltpu.VMEM((2, QCHUNK, 16), jnp.float32),
            pltpu.VMEM((G, C // 2), jnp.int32),
            pltpu.VMEM((G, C // 2), jnp.int32),
            pltpu.VMEM((C,), jnp.float32),
            pltpu.VMEM((C,), jnp.float32),
            pltpu.VMEM((C,), jnp.float32),
            pltpu.VMEM((C,), jnp.float32),
            pltpu.SemaphoreType.DMA,
            pltpu.SemaphoreType.DMA,
            pltpu.SemaphoreType.DMA,
            pltpu.SemaphoreType.DMA,
        ],
    )
    return kern(table, idx_q, wgt_q, r_q, op_b, gvec)


@jax.jit
def kernel(query, ref_points, image_features, mask, dp_w, dp_b, A_w, A_b,
           W_w, W_b, vp_w, vp_b, op_w, op_b):
    f32 = jnp.float32

    w_aug = jnp.pad(jnp.concatenate([W_w.T, W_b[None, :]], axis=0),
                    ((0, 7), (0, 0)))
    vp_aug = jnp.pad(jnp.concatenate([vp_w.T, vp_b[None, :]], axis=0),
                     ((0, 7), (0, 0)))
    t1_aug, g_aug = _fuse_weights(w_aug, op_w.T, vp_aug)
    gvec = t1_aug[C]

    perm = np.arange(C).reshape(C // 32, 2, 16)
    perm = np.stack([perm[:, 0], perm[:, 1]], axis=-1).reshape(C)
    x = image_features.transpose(0, 2, 3, 1).reshape(V, C)
    table = _make_table(x, g_aug[:, perm])
    table = lax.bitcast_convert_type(table.reshape(V, C // 2, 2), jnp.int32)

    q_t = jnp.pad(query.transpose(0, 2, 1), ((0, 0), (0, 0), (0, NQPAD - NQ)))
    refx = jnp.pad(ref_points[..., 0], ((0, 0), (0, 2), (0, NQPAD - NQ)))
    refy = jnp.pad(ref_points[..., 1], ((0, 0), (0, 2), (0, NQPAD - NQ)))
    mask_t = jnp.pad(mask, ((0, 0), (0, 2), (0, NQPAD - NQ)))
    dpx_w = dp_w[0::2]
    dpy_w = dp_w[1::2]
    dpb = jnp.broadcast_to(
        jnp.concatenate([dp_b[0::2], dp_b[1::2]])[:, None], (2 * Ns, 128))
    ab = jnp.broadcast_to(A_b[:, None], (Ns, 128))

    idx_t, wgt_t, r_t = _prep(q_t, refx, refy, mask_t, dpx_w, dpy_w, A_w,
                              dpb, ab)

    idx_q = idx_t.transpose(0, 2, 1).reshape(BTOT, 2, G // 2)
    wgt_q = wgt_t.transpose(0, 2, 1).reshape(BTOT, G)
    r_q = r_t.transpose(0, 2, 1).reshape(BTOT, 16)

    z = _sc_gather_reduce(table, idx_q, wgt_q, r_q,
                          op_b.astype(f32), gvec)
    return z.reshape(B, NQPAD, C)[:, :NQ]


# speedup vs baseline: 1.9176x; 1.2881x over previous
import functools
import jax
import jax.numpy as jnp
import numpy as np
from jax import lax
from jax.experimental import pallas as pl
from jax.experimental.pallas import tpu as pltpu
from jax.experimental.pallas import tpu_sc as plsc

B, N, NQ, C, Ns, H, W = 2, 6, 2500, 256, 8, 64, 64
BN = B * N
V = BN * H * W
G = N * Ns * 4
QB = 256
NQPAD = 2560
BTOT = B * NQPAD
NWORK = 32
QPW = BTOT // NWORK
QCHUNK = 16


def _fuse_body(w_aug_ref, op_t_ref, vp_aug_ref, t1_ref, g_ref):
    t1 = jnp.dot(w_aug_ref[...], op_t_ref[...], preferred_element_type=jnp.float32)
    t1_ref[...] = t1
    g_ref[...] = jnp.dot(vp_aug_ref[...], t1[0:C, :], preferred_element_type=jnp.float32)


def _fuse_weights(w_aug, op_t, vp_aug):
    return pl.pallas_call(
        _fuse_body,
        out_shape=(
            jax.ShapeDtypeStruct((264, C), jnp.float32),
            jax.ShapeDtypeStruct((264, C), jnp.float32),
        ),
    )(w_aug, op_t, vp_aug)


def _table_body(x_ref, ga_ref, o_ref):
    o_ref[...] = (
        jnp.dot(x_ref[...], ga_ref[0:C, :], preferred_element_type=jnp.float32)
        + ga_ref[C:C + 1, :]
    ).astype(jnp.bfloat16)


def _make_table(x, g_aug):
    blk = 1024
    return pl.pallas_call(
        _table_body,
        grid=(V // blk,),
        in_specs=[
            pl.BlockSpec((blk, C), lambda i: (i, 0)),
            pl.BlockSpec((264, C), lambda i: (0, 0)),
        ],
        out_specs=pl.BlockSpec((blk, C), lambda i: (i, 0)),
        out_shape=jax.ShapeDtypeStruct((V, C), jnp.bfloat16),
    )(x, g_aug)


def _prep_body(q_ref, refx_ref, refy_ref, mask_ref, dpx_w_ref, dpy_w_ref,
               a_w_ref, dpb_ref, ab_ref, idx_ref, wgt_ref, r_ref):
    b = pl.program_id(0)
    qb = q_ref[0]
    dpx = jnp.dot(dpx_w_ref[...], qb, preferred_element_type=jnp.float32)
    dpx = dpx + dpb_ref[0:Ns, 0:1]
    dpy = jnp.dot(dpy_w_ref[...], qb, preferred_element_type=jnp.float32)
    dpy = dpy + dpb_ref[Ns:2 * Ns, 0:1]
    logits = jnp.dot(a_w_ref[...], qb, preferred_element_type=jnp.float32)
    logits = logits + ab_ref[:, 0:1]
    mx = jnp.max(logits, axis=0, keepdims=True)
    ex = jnp.exp(logits - mx)
    attn = ex / jnp.sum(ex, axis=0, keepdims=True)

    msum = jnp.sum(mask_ref[0], axis=0, keepdims=True)
    r = msum / (msum + 1e-6)
    r_ref[0] = jnp.broadcast_to(r, (16, r.shape[1]))
    inv_m = 1.0 / (msum + 1e-6)

    for n in range(N):
        mrow = mask_ref[0, n:n + 1, :]
        px = (refx_ref[0, n:n + 1, :] + dpx) * (W - 1.0)
        py = (refy_ref[0, n:n + 1, :] + dpy) * (H - 1.0)
        x0 = jnp.floor(px)
        y0 = jnp.floor(py)
        fx = px - x0
        fy = py - y0
        wq = mrow * attn * inv_m
        base = (b * N + n) * (H * W)
        ci = 0
        for dy, wyf in ((0, 1.0 - fy), (1, fy)):
            for dx, wxf in ((0, 1.0 - fx), (1, fx)):
                xi = x0 + dx
                yi = y0 + dy
                valid = ((xi >= 0.0) & (xi <= W - 1.0)
                         & (yi >= 0.0) & (yi <= H - 1.0))
                xc = jnp.clip(xi, 0.0, W - 1.0).astype(jnp.int32)
                yc = jnp.clip(yi, 0.0, H - 1.0).astype(jnp.int32)
                sub = n * (4 * Ns) + ci * Ns
                idx_ref[0, sub:sub + Ns, :] = base + yc * W + xc
                wgt_ref[0, sub:sub + Ns, :] = wq * wxf * wyf * valid.astype(jnp.float32)
                ci += 1


def _prep(q_t, refx, refy, mask_t, dpx_w, dpy_w, a_w, dpb, ab):
    nb = NQPAD // QB
    return pl.pallas_call(
        _prep_body,
        grid=(B, nb),
        in_specs=[
            pl.BlockSpec((1, C, QB), lambda b, j: (b, 0, j)),
            pl.BlockSpec((1, 8, QB), lambda b, j: (b, 0, j)),
            pl.BlockSpec((1, 8, QB), lambda b, j: (b, 0, j)),
            pl.BlockSpec((1, 8, QB), lambda b, j: (b, 0, j)),
            pl.BlockSpec((Ns, C), lambda b, j: (0, 0)),
            pl.BlockSpec((Ns, C), lambda b, j: (0, 0)),
            pl.BlockSpec((Ns, C), lambda b, j: (0, 0)),
            pl.BlockSpec((2 * Ns, 128), lambda b, j: (0, 0)),
            pl.BlockSpec((Ns, 128), lambda b, j: (0, 0)),
        ],
        out_specs=[
            pl.BlockSpec((1, G, QB), lambda b, j: (b, 0, j)),
            pl.BlockSpec((1, G, QB), lambda b, j: (b, 0, j)),
            pl.BlockSpec((1, 16, QB), lambda b, j: (b, 0, j)),
        ],
        out_shape=(
            jax.ShapeDtypeStruct((B, G, NQPAD), jnp.int32),
            jax.ShapeDtypeStruct((B, G, NQPAD), jnp.float32),
            jax.ShapeDtypeStruct((B, 16, NQPAD), jnp.float32),
        ),
    )(q_t, refx, refy, mask_t, dpx_w, dpy_w, a_w, dpb, ab)


NCHUNK = 16


def _sc_body(table_hbm, idx_hbm, wgt_hbm, r_hbm, opb_hbm, gv_hbm, out_hbm,
             meta_i, meta_w, meta_r, rows0, rows1, acc0, acc1, opb_v, gv_v,
             sem_g0, sem_g1, sem_s0, sem_s1):
    wid = lax.axis_index("s") * 2 + lax.axis_index("c")
    qbase = wid * QPW
    pltpu.sync_copy(opb_hbm, opb_v)
    pltpu.sync_copy(gv_hbm, gv_v)

    def copy_meta(i):
        blk = i >> 4
        slot = blk & 1
        q0 = qbase + blk * QCHUNK
        pltpu.sync_copy(idx_hbm.at[pl.ds(q0, QCHUNK)], meta_i.at[slot])
        pltpu.sync_copy(wgt_hbm.at[pl.ds(q0, QCHUNK)], meta_w.at[slot])
        pltpu.sync_copy(r_hbm.at[pl.ds(q0, QCHUNK)], meta_r.at[slot])

    def gather_cps(i, rows, sem):
        slot = (i >> 4) & 1
        mi = i & 15
        cpa = pltpu.make_async_copy(table_hbm.at[meta_i.at[slot, mi, 0]],
                                    rows.at[pl.ds(0, G // 2)], sem)
        cpb = pltpu.make_async_copy(table_hbm.at[meta_i.at[slot, mi, 1]],
                                    rows.at[pl.ds(G // 2, G // 2)], sem)
        return cpa, cpb

    def reduce_to(i, rows, acc):
        slot = (i >> 4) & 1
        mi = i & 15
        rv = meta_r[slot, mi, pl.ds(0, 16)][0]
        for h in range(4):

            def red(j, accs):
                wvec = meta_w[slot, mi, pl.ds(j * 16, 16)]
                rbase = j * 16
                for e in range(16):
                    we = wvec[e]
                    new = list(accs)
                    for g2 in range(2):
                        grp = h * 2 + g2
                        v = rows[rbase + e, pl.ds(grp * 16, 16)]
                        lo = lax.bitcast_convert_type(v << 16, jnp.float32)
                        hi = lax.bitcast_convert_type(
                            v & jnp.int32(-65536), jnp.float32)
                        new[2 * g2] = new[2 * g2] + lo * we
                        new[2 * g2 + 1] = new[2 * g2 + 1] + hi * we
                    accs = tuple(new)
                return accs

            accs = lax.fori_loop(
                0, G // 16, red,
                tuple(jnp.zeros((16,), jnp.float32) for _ in range(4)))
            for g2 in range(2):
                grp = h * 2 + g2
                for half in range(2):
                    sl = pl.ds(grp * 32 + half * 16, 16)
                    acc[sl] = (accs[2 * g2 + half] + opb_v[sl]
                               + gv_v[sl] * rv)

    copy_meta(0)
    pa, pb = gather_cps(0, rows0, sem_g0)
    pa.start()
    pb.start()

    nk = QPW // 2

    def k_body(k, _):
        i0 = 2 * k
        i1 = 2 * k + 1
        wa, wb = gather_cps(i0, rows0, sem_g0)
        wa.wait()
        wb.wait()
        c1a, c1b = gather_cps(i1, rows1, sem_g1)
        c1a.start()
        c1b.start()
        reduce_to(i0, rows0, acc0)

        @pl.when(k > 0)
        def _w0():
            pltpu.make_async_copy(acc0, out_hbm.at[qbase + i0 - 2],
                                  sem_s0).wait()

        pltpu.make_async_copy(acc0, out_hbm.at[qbase + i0], sem_s0).start()

        c1a.wait()
        c1b.wait()

        @pl.when(k < nk - 1)
        def _nx():
            inext = i0 + 2

            @pl.when((inext & 15) == 0)
            def _cm():
                copy_meta(inext)

            na, nb = gather_cps(inext, rows0, sem_g0)
            na.start()
            nb.start()

        reduce_to(i1, rows1, acc1)

        @pl.when(k > 0)
        def _w1():
            pltpu.make_async_copy(acc1, out_hbm.at[qbase + i1 - 2],
                                  sem_s1).wait()

        pltpu.make_async_copy(acc1, out_hbm.at[qbase + i1], sem_s1).start()
        return _

    lax.fori_loop(0, nk, k_body, 0)
    pltpu.make_async_copy(acc0, out_hbm.at[qbase + QPW - 2], sem_s0).wait()
    pltpu.make_async_copy(acc1, out_hbm.at[qbase + QPW - 1], sem_s1).wait()


def _sc_gather_reduce(table, idx_q, wgt_q, r_q, op_b, gvec):
    kern = pl.kernel(
        _sc_body,
        out_type=jax.ShapeDtypeStruct((BTOT, C), jnp.float32),
        mesh=plsc.VectorSubcoreMesh(core_axis_name="c", subcore_axis_name="s"),
        scratch_types=[
            pltpu.VMEM((2, QCHUNK, 2, G // 2), jnp.int32),
            pltpu.VMEM((2, QCHUNK, G), jnp.float32),
            pltpu.VMEM((2, QCHUNK, 16), jnp.float32),
            pltpu.VMEM((G, C // 2), jnp.int32),
            pltpu.VMEM((G, C // 2), jnp.int32),
            pltpu.VMEM((C,), jnp.float32),
            pltpu.VMEM((C,), jnp.float32),
            pltpu.VMEM((C,), jnp.float32),
            pltpu.VMEM((C,), jnp.float32),
            pltpu.SemaphoreType.DMA,
            pltpu.SemaphoreType.DMA,
            pltpu.SemaphoreType.DMA,
            pltpu.SemaphoreType.DMA,
        ],
    )
    return kern(table, idx_q, wgt_q, r_q, op_b, gvec)


@jax.jit
def kernel(query, ref_points, image_features, mask, dp_w, dp_b, A_w, A_b,
           W_w, W_b, vp_w, vp_b, op_w, op_b):
    f32 = jnp.float32

    w_aug = jnp.pad(jnp.concatenate([W_w.T, W_b[None, :]], axis=0),
                    ((0, 7), (0, 0)))
    vp_aug = jnp.pad(jnp.concatenate([vp_w.T, vp_b[None, :]], axis=0),
                     ((0, 7), (0, 0)))
    t1_aug, g_aug = _fuse_weights(w_aug, op_w.T, vp_aug)
    gvec = t1_aug[C]

    perm = np.arange(C).reshape(C // 32, 2, 16)
    perm = np.stack([perm[:, 0], perm[:, 1]], axis=-1).reshape(C)
    x = image_features.transpose(0, 2, 3, 1).reshape(V, C)
    table = _make_table(x, g_aug[:, perm])
    table = lax.bitcast_convert_type(table.reshape(V, C // 2, 2), jnp.int32)

    q_t = jnp.pad(query.transpose(0, 2, 1), ((0, 0), (0, 0), (0, NQPAD - NQ)))
    refx = jnp.pad(ref_points[..., 0], ((0, 0), (0, 2), (0, NQPAD - NQ)))
    refy = jnp.pad(ref_points[..., 1], ((0, 0), (0, 2), (0, NQPAD - NQ)))
    mask_t = jnp.pad(mask, ((0, 0), (0, 2), (0, NQPAD - NQ)))
    dpx_w = dp_w[0::2]
    dpy_w = dp_w[1::2]
    dpb = jnp.broadcast_to(
        jnp.concatenate([dp_b[0::2], dp_b[1::2]])[:, None], (2 * Ns, 128))
    ab = jnp.broadcast_to(A_b[:, None], (Ns, 128))

    idx_t, wgt_t, r_t = _prep(q_t, refx, refy, mask_t, dpx_w, dpy_w, A_w,
                              dpb, ab)

    idx_q = idx_t.transpose(0, 2, 1).reshape(BTOT, 2, G // 2)
    wgt_q = wgt_t.transpose(0, 2, 1).reshape(BTOT, G)
    r_q = r_t.transpose(0, 2, 1).reshape(BTOT, 16)

    z = _sc_gather_reduce(table, idx_q, wgt_q, r_q,
                          op_b.astype(f32), gvec)
    return z.reshape(B, NQPAD, C)[:, :NQ]


# speedup vs baseline: 2.6518x; 1.3829x over previous
import functools
import jax
import jax.numpy as jnp
import numpy as np
from jax import lax
from jax.experimental import pallas as pl
from jax.experimental.pallas import tpu as pltpu
from jax.experimental.pallas import tpu_sc as plsc

B, N, NQ, C, Ns, H, W = 2, 6, 2500, 256, 8, 64, 64
BN = B * N
V = BN * H * W
G = N * Ns * 4
QB = 256
NQPAD = 2560
BTOT = B * NQPAD
NWORK = 32
QPW = BTOT // NWORK
QCHUNK = 16


def _fuse_body(w_aug_ref, op_t_ref, vp_aug_ref, t1_ref, g_ref):
    t1 = jnp.dot(w_aug_ref[...], op_t_ref[...], preferred_element_type=jnp.float32)
    t1_ref[...] = t1
    g_ref[...] = jnp.dot(vp_aug_ref[...], t1[0:C, :], preferred_element_type=jnp.float32)


def _fuse_weights(w_aug, op_t, vp_aug):
    return pl.pallas_call(
        _fuse_body,
        out_shape=(
            jax.ShapeDtypeStruct((264, C), jnp.float32),
            jax.ShapeDtypeStruct((264, C), jnp.float32),
        ),
    )(w_aug, op_t, vp_aug)


def _table_body(x_ref, ga_ref, o_ref):
    o_ref[...] = (
        jnp.dot(x_ref[...], ga_ref[0:C, :], preferred_element_type=jnp.float32)
        + ga_ref[C:C + 1, :]
    )


def _make_table(x, g_aug):
    blk = 1024
    return pl.pallas_call(
        _table_body,
        grid=(V // blk,),
        in_specs=[
            pl.BlockSpec((blk, C), lambda i: (i, 0)),
            pl.BlockSpec((264, C), lambda i: (0, 0)),
        ],
        out_specs=pl.BlockSpec((blk, C), lambda i: (i, 0)),
        out_shape=jax.ShapeDtypeStruct((V, C), jnp.float32),
    )(x, g_aug)


def _prep_body(q_ref, refx_ref, refy_ref, mask_ref, dpx_w_ref, dpy_w_ref,
               a_w_ref, dpb_ref, ab_ref, idx_ref, wgt_ref, r_ref):
    b = pl.program_id(0)
    qb = q_ref[0]
    dpx = jnp.dot(dpx_w_ref[...], qb, preferred_element_type=jnp.float32)
    dpx = dpx + dpb_ref[0:Ns, 0:1]
    dpy = jnp.dot(dpy_w_ref[...], qb, preferred_element_type=jnp.float32)
    dpy = dpy + dpb_ref[Ns:2 * Ns, 0:1]
    logits = jnp.dot(a_w_ref[...], qb, preferred_element_type=jnp.float32)
    logits = logits + ab_ref[:, 0:1]
    mx = jnp.max(logits, axis=0, keepdims=True)
    ex = jnp.exp(logits - mx)
    attn = ex / jnp.sum(ex, axis=0, keepdims=True)

    msum = jnp.sum(mask_ref[0], axis=0, keepdims=True)
    r = msum / (msum + 1e-6)
    r_ref[0] = jnp.broadcast_to(r, (16, r.shape[1]))
    inv_m = 1.0 / (msum + 1e-6)

    for n in range(N):
        mrow = mask_ref[0, n:n + 1, :]
        px = (refx_ref[0, n:n + 1, :] + dpx) * (W - 1.0)
        py = (refy_ref[0, n:n + 1, :] + dpy) * (H - 1.0)
        x0 = jnp.floor(px)
        y0 = jnp.floor(py)
        fx = px - x0
        fy = py - y0
        wq = mrow * attn * inv_m
        base = (b * N + n) * (H * W)
        ci = 0
        for dy, wyf in ((0, 1.0 - fy), (1, fy)):
            for dx, wxf in ((0, 1.0 - fx), (1, fx)):
                xi = x0 + dx
                yi = y0 + dy
                valid = ((xi >= 0.0) & (xi <= W - 1.0)
                         & (yi >= 0.0) & (yi <= H - 1.0))
                xc = jnp.clip(xi, 0.0, W - 1.0).astype(jnp.int32)
                yc = jnp.clip(yi, 0.0, H - 1.0).astype(jnp.int32)
                sub = n * (4 * Ns) + ci * Ns
                idx_ref[0, sub:sub + Ns, :] = base + yc * W + xc
                wgt_ref[0, sub:sub + Ns, :] = wq * wxf * wyf * valid.astype(jnp.float32)
                ci += 1


def _prep(q_t, refx, refy, mask_t, dpx_w, dpy_w, a_w, dpb, ab):
    nb = NQPAD // QB
    return pl.pallas_call(
        _prep_body,
        grid=(B, nb),
        in_specs=[
            pl.BlockSpec((1, C, QB), lambda b, j: (b, 0, j)),
            pl.BlockSpec((1, 8, QB), lambda b, j: (b, 0, j)),
            pl.BlockSpec((1, 8, QB), lambda b, j: (b, 0, j)),
            pl.BlockSpec((1, 8, QB), lambda b, j: (b, 0, j)),
            pl.BlockSpec((Ns, C), lambda b, j: (0, 0)),
            pl.BlockSpec((Ns, C), lambda b, j: (0, 0)),
            pl.BlockSpec((Ns, C), lambda b, j: (0, 0)),
            pl.BlockSpec((2 * Ns, 128), lambda b, j: (0, 0)),
            pl.BlockSpec((Ns, 128), lambda b, j: (0, 0)),
        ],
        out_specs=[
            pl.BlockSpec((1, G, QB), lambda b, j: (b, 0, j)),
            pl.BlockSpec((1, G, QB), lambda b, j: (b, 0, j)),
            pl.BlockSpec((1, 16, QB), lambda b, j: (b, 0, j)),
        ],
        out_shape=(
            jax.ShapeDtypeStruct((B, G, NQPAD), jnp.int32),
            jax.ShapeDtypeStruct((B, G, NQPAD), jnp.float32),
            jax.ShapeDtypeStruct((B, 16, NQPAD), jnp.float32),
        ),
    )(q_t, refx, refy, mask_t, dpx_w, dpy_w, a_w, dpb, ab)


NCHUNK = 16


def _sc_body(table_hbm, idx_hbm, wgt_hbm, r_hbm, opb_hbm, gv_hbm, out_hbm,
             meta_i, meta_w, meta_r, rows0, rows1, acc0, acc1, opb_v, gv_v,
             sem_g0, sem_g1, sem_s0, sem_s1):
    wid = lax.axis_index("s") * 2 + lax.axis_index("c")
    qbase = wid * QPW
    pltpu.sync_copy(opb_hbm, opb_v)
    pltpu.sync_copy(gv_hbm, gv_v)

    def copy_meta(i):
        blk = i >> 4
        slot = blk & 1
        q0 = qbase + blk * QCHUNK
        pltpu.sync_copy(idx_hbm.at[pl.ds(q0, QCHUNK)], meta_i.at[slot])
        pltpu.sync_copy(wgt_hbm.at[pl.ds(q0, QCHUNK)], meta_w.at[slot])
        pltpu.sync_copy(r_hbm.at[pl.ds(q0, QCHUNK)], meta_r.at[slot])

    def gather_cps(i, rows, sem):
        slot = (i >> 4) & 1
        mi = i & 15
        cpa = pltpu.make_async_copy(table_hbm.at[meta_i.at[slot, mi, 0]],
                                    rows.at[pl.ds(0, G // 2)], sem)
        cpb = pltpu.make_async_copy(table_hbm.at[meta_i.at[slot, mi, 1]],
                                    rows.at[pl.ds(G // 2, G // 2)], sem)
        return cpa, cpb

    def reduce_to(i, rows, acc):
        slot = (i >> 4) & 1
        mi = i & 15
        rv = meta_r[slot, mi, pl.ds(0, 16)][0]
        for h in range(2):

            def red(j, accs):
                wvec = meta_w[slot, mi, pl.ds(j * 16, 16)]
                rbase = j * 16
                for e in range(16):
                    we = wvec[e]
                    new = list(accs)
                    for c8 in range(8):
                        c = h * 8 + c8
                        new[c8] = (new[c8]
                                   + rows[rbase + e, pl.ds(c * 16, 16)] * we)
                    accs = tuple(new)
                return accs

            accs = lax.fori_loop(
                0, G // 16, red,
                tuple(jnp.zeros((16,), jnp.float32) for _ in range(8)))
            for c8 in range(8):
                sl = pl.ds((h * 8 + c8) * 16, 16)
                acc[sl] = accs[c8] + opb_v[sl] + gv_v[sl] * rv

    copy_meta(0)
    pa, pb = gather_cps(0, rows0, sem_g0)
    pa.start()
    pb.start()

    nk = QPW // 2

    def k_body(k, _):
        i0 = 2 * k
        i1 = 2 * k + 1
        wa, wb = gather_cps(i0, rows0, sem_g0)
        wa.wait()
        wb.wait()
        c1a, c1b = gather_cps(i1, rows1, sem_g1)
        c1a.start()
        c1b.start()
        reduce_to(i0, rows0, acc0)

        @pl.when(k > 0)
        def _w0():
            pltpu.make_async_copy(acc0, out_hbm.at[qbase + i0 - 2],
                                  sem_s0).wait()

        pltpu.make_async_copy(acc0, out_hbm.at[qbase + i0], sem_s0).start()

        c1a.wait()
        c1b.wait()

        @pl.when(k < nk - 1)
        def _nx():
            inext = i0 + 2

            @pl.when((inext & 15) == 0)
            def _cm():
                copy_meta(inext)

            na, nb = gather_cps(inext, rows0, sem_g0)
            na.start()
            nb.start()

        reduce_to(i1, rows1, acc1)

        @pl.when(k > 0)
        def _w1():
            pltpu.make_async_copy(acc1, out_hbm.at[qbase + i1 - 2],
                                  sem_s1).wait()

        pltpu.make_async_copy(acc1, out_hbm.at[qbase + i1], sem_s1).start()
        return _

    lax.fori_loop(0, nk, k_body, 0)
    pltpu.make_async_copy(acc0, out_hbm.at[qbase + QPW - 2], sem_s0).wait()
    pltpu.make_async_copy(acc1, out_hbm.at[qbase + QPW - 1], sem_s1).wait()


def _sc_gather_reduce(table, idx_q, wgt_q, r_q, op_b, gvec):
    kern = pl.kernel(
        _sc_body,
        out_type=jax.ShapeDtypeStruct((BTOT, C), jnp.float32),
        mesh=plsc.VectorSubcoreMesh(core_axis_name="c", subcore_axis_name="s"),
        scratch_types=[
            pltpu.VMEM((2, QCHUNK, 2, G // 2), jnp.int32),
            pltpu.VMEM((2, QCHUNK, G), jnp.float32),
            pltpu.VMEM((2, QCHUNK, 16), jnp.float32),
            pltpu.VMEM((G, C), jnp.float32),
            pltpu.VMEM((G, C), jnp.float32),
            pltpu.VMEM((C,), jnp.float32),
            pltpu.VMEM((C,), jnp.float32),
            pltpu.VMEM((C,), jnp.float32),
            pltpu.VMEM((C,), jnp.float32),
            pltpu.SemaphoreType.DMA,
            pltpu.SemaphoreType.DMA,
            pltpu.SemaphoreType.DMA,
            pltpu.SemaphoreType.DMA,
        ],
    )
    return kern(table, idx_q, wgt_q, r_q, op_b, gvec)


@jax.jit
def kernel(query, ref_points, image_features, mask, dp_w, dp_b, A_w, A_b,
           W_w, W_b, vp_w, vp_b, op_w, op_b):
    f32 = jnp.float32

    w_aug = jnp.pad(jnp.concatenate([W_w.T, W_b[None, :]], axis=0),
                    ((0, 7), (0, 0)))
    vp_aug = jnp.pad(jnp.concatenate([vp_w.T, vp_b[None, :]], axis=0),
                     ((0, 7), (0, 0)))
    t1_aug, g_aug = _fuse_weights(w_aug, op_w.T, vp_aug)
    gvec = t1_aug[C]

    x = image_features.transpose(0, 2, 3, 1).reshape(V, C)
    table = _make_table(x, g_aug)

    q_t = jnp.pad(query.transpose(0, 2, 1), ((0, 0), (0, 0), (0, NQPAD - NQ)))
    refx = jnp.pad(ref_points[..., 0], ((0, 0), (0, 2), (0, NQPAD - NQ)))
    refy = jnp.pad(ref_points[..., 1], ((0, 0), (0, 2), (0, NQPAD - NQ)))
    mask_t = jnp.pad(mask, ((0, 0), (0, 2), (0, NQPAD - NQ)))
    dpx_w = dp_w[0::2]
    dpy_w = dp_w[1::2]
    dpb = jnp.broadcast_to(
        jnp.concatenate([dp_b[0::2], dp_b[1::2]])[:, None], (2 * Ns, 128))
    ab = jnp.broadcast_to(A_b[:, None], (Ns, 128))

    idx_t, wgt_t, r_t = _prep(q_t, refx, refy, mask_t, dpx_w, dpy_w, A_w,
                              dpb, ab)

    idx_q = idx_t.transpose(0, 2, 1).reshape(BTOT, 2, G // 2)
    wgt_q = wgt_t.transpose(0, 2, 1).reshape(BTOT, G)
    r_q = r_t.transpose(0, 2, 1).reshape(BTOT, 16)

    z = _sc_gather_reduce(table, idx_q, wgt_q, r_q,
                          op_b.astype(f32), gvec)
    return z.reshape(B, NQPAD, C)[:, :NQ]
